# Initial kernel scaffold; baseline (speedup 1.0000x reference)
import functools

import jax
import jax.numpy as jnp
from jax import lax
from jax.experimental import pallas as pl
from jax.experimental.pallas import tpu as pltpu
from jax.experimental.pallas import tpu_sc as plsc

N = 50000
E = 800000
D_IN = 11
H = 128
C = 19
G = 64

NPAD = 50048
SLAB = NPAD // 16
EPAD = 802816
EROWS = EPAD // 128
BLK = 2176
NB = NPAD // BLK

_MESH = dict(core_axis_name="c", subcore_axis_name="s", num_cores=2,
             num_subcores=16)

f32 = jnp.float32
i32 = jnp.int32


def _mesh():
    return plsc.VectorSubcoreMesh(**_MESH)


def _deg_call(dst2d, ones_h, zer1_h):
    @functools.partial(
        pl.kernel,
        out_type=(jax.ShapeDtypeStruct((NPAD,), f32),
                  jax.ShapeDtypeStruct((NPAD,), f32)),
        mesh=_mesh(),
        scratch_types=[pltpu.VMEM((14, 128), i32),
                       pltpu.VMEM((128,), f32),
                       pltpu.VMEM_SHARED((NPAD,), f32)],
    )
    def k(dst_h, one_h, z_h, out0, out1, didx, ones_v, acc):
        c = lax.axis_index("c")
        s = lax.axis_index("s")
        w = c * 16 + s
        pltpu.sync_copy(one_h, ones_v)
        pltpu.sync_copy(z_h, acc.at[pl.ds(s * SLAB, SLAB)])
        plsc.subcore_barrier()

        def body(i, _):
            rb = w * 196 + i * 14
            pltpu.sync_copy(dst_h.at[pl.ds(rb, 14), :], didx)
            for j in range(14):
                pltpu.sync_copy(ones_v, acc.at[didx.at[j]], add=True)
            return 0

        lax.fori_loop(0, 14, body, 0)
        plsc.subcore_barrier()
        sl = pl.ds(s * SLAB, SLAB)

        @pl.when(c == 0)
        def _():
            pltpu.sync_copy(acc.at[sl], out0.at[sl])

        @pl.when(c == 1)
        def _():
            pltpu.sync_copy(acc.at[sl], out1.at[sl])

    return k(dst2d, ones_h, zer1_h)


def _prop16_call(src2d, dst2d, xt, zer16_h):
    @functools.partial(
        pl.kernel,
        out_type=(jax.ShapeDtypeStruct((NPAD, 16), f32),
                  jax.ShapeDtypeStruct((NPAD, 16), f32)),
        mesh=_mesh(),
        scratch_types=[pltpu.VMEM((14, 128), i32),
                       pltpu.VMEM((14, 128), i32),
                       pltpu.VMEM((128, 16), f32),
                       pltpu.VMEM_SHARED((NPAD, 16), f32)],
    )
    def k(src_h, dst_h, x_h, z_h, out0, out1, sidx, didx, rows, acc):
        c = lax.axis_index("c")
        s = lax.axis_index("s")
        w = c * 16 + s
        pltpu.sync_copy(z_h, acc.at[pl.ds(s * SLAB, SLAB), :])
        plsc.subcore_barrier()

        def body(i, _):
            rb = w * 196 + i * 14
            pltpu.sync_copy(src_h.at[pl.ds(rb, 14), :], sidx)
            pltpu.sync_copy(dst_h.at[pl.ds(rb, 14), :], didx)
            for j in range(14):
                pltpu.sync_copy(x_h.at[sidx.at[j]], rows)
                pltpu.sync_copy(rows, acc.at[didx.at[j]], add=True)
            return 0

        lax.fori_loop(0, 14, body, 0)
        plsc.subcore_barrier()
        sl = pl.ds(s * SLAB, SLAB)

        @pl.when(c == 0)
        def _():
            pltpu.sync_copy(acc.at[sl, :], out0.at[sl, :])

        @pl.when(c == 1)
        def _():
            pltpu.sync_copy(acc.at[sl, :], out1.at[sl, :])

    return k(src2d, dst2d, xt, zer16_h)


def _prop32_call(src2d, dst2d, q0, q1, q2, q3, zer32_h):
    @functools.partial(
        pl.kernel,
        out_type=tuple(jax.ShapeDtypeStruct((NPAD, 32), f32)
                       for _ in range(4)),
        mesh=_mesh(),
        scratch_types=[pltpu.VMEM((14, 128), i32),
                       pltpu.VMEM((14, 128), i32),
                       pltpu.VMEM((128, 32), f32),
                       pltpu.VMEM_SHARED((NPAD, 32), f32)],
    )
    def k(src_h, dst_h, x0, x1, x2, x3, z_h,
          o0, o1, o2, o3, sidx, didx, rows, acc):
        c = lax.axis_index("c")
        s = lax.axis_index("s")
        sl = pl.ds(s * SLAB, SLAB)

        def qpass(x_h, o_h):
            pltpu.sync_copy(z_h, acc.at[sl, :])
            plsc.subcore_barrier()

            def body(i, _):
                rb = s * 392 + i * 14
                pltpu.sync_copy(src_h.at[pl.ds(rb, 14), :], sidx)
                pltpu.sync_copy(dst_h.at[pl.ds(rb, 14), :], didx)
                for j in range(14):
                    pltpu.sync_copy(x_h.at[sidx.at[j]], rows)
                    pltpu.sync_copy(rows, acc.at[didx.at[j]], add=True)
                return 0

            lax.fori_loop(0, 28, body, 0)
            plsc.subcore_barrier()
            pltpu.sync_copy(acc.at[sl, :], o_h.at[sl, :])

        @pl.when(c == 0)
        def _():
            qpass(x0, o0)
            qpass(x1, o1)

        @pl.when(c == 1)
        def _():
            qpass(x2, o2)
            qpass(x3, o3)

    return k(src2d, dst2d, q0, q1, q2, q3, zer32_h)


def _full(shape):
    return pl.BlockSpec(shape, lambda i: (0,) * len(shape))


def _rows(w):
    return pl.BlockSpec((BLK, w), lambda i: (i, 0))


def _tc_scale_call(d0, d1, xpad):
    def body(d0_r, d1_r, x_r, dis_r, xt_r):
        dis = lax.rsqrt(d0_r[...] + d1_r[...] + 1.0)
        dis_r[...] = dis
        xt_r[...] = x_r[...] * dis

    return pl.pallas_call(
        body,
        grid=(NB,),
        in_specs=[_rows(1), _rows(1), _rows(16)],
        out_specs=[_rows(1), _rows(16)],
        out_shape=[jax.ShapeDtypeStruct((NPAD, 1), f32),
                   jax.ShapeDtypeStruct((NPAD, 16), f32)],
    )(d0, d1, xpad)


def _tc_layer1_call(p0, p1, xt, dis, W1p, b1r, W2):
    def body(p0_r, p1_r, xt_r, dis_r, w1_r, b1_r, w2_r, o0, o1, o2, o3):
        dis = dis_r[...]
        agg = (p0_r[...] + p1_r[...] + xt_r[...]) * dis
        h1 = jnp.maximum(
            jnp.dot(agg, w1_r[...], preferred_element_type=f32) + b1_r[...],
            0.0)
        g = jnp.dot(h1, w2_r[...], preferred_element_type=f32) * dis
        o0[...] = g[:, 0:32]
        o1[...] = g[:, 32:64]
        o2[...] = g[:, 64:96]
        o3[...] = g[:, 96:128]

    return pl.pallas_call(
        body,
        grid=(NB,),
        in_specs=[_rows(16), _rows(16), _rows(16), _rows(1),
                  _full((16, H)), _full((1, H)), _full((H, H))],
        out_specs=[_rows(32)] * 4,
        out_shape=[jax.ShapeDtypeStruct((NPAD, 32), f32)] * 4,
    )(p0, p1, xt, dis, W1p, b1r, W2)


def _tc_mid_call(s0, s1, s2, s3, q0, q1, q2, q3, dis, br, W):
    def body(s0_r, s1_r, s2_r, s3_r, q0_r, q1_r, q2_r, q3_r, dis_r, b_r,
             w_r, o0, o1, o2, o3):
        dis = dis_r[...]
        t = jnp.concatenate(
            [s0_r[...] + q0_r[...], s1_r[...] + q1_r[...],
             s2_r[...] + q2_r[...], s3_r[...] + q3_r[...]], axis=1)
        h = jnp.maximum(t * dis + b_r[...], 0.0)
        g = jnp.dot(h, w_r[...], preferred_element_type=f32) * dis
        o0[...] = g[:, 0:32]
        o1[...] = g[:, 32:64]
        o2[...] = g[:, 64:96]
        o3[...] = g[:, 96:128]

    return pl.pallas_call(
        body,
        grid=(NB,),
        in_specs=[_rows(32)] * 8 + [_rows(1), _full((1, H)), _full((H, H))],
        out_specs=[_rows(32)] * 4,
        out_shape=[jax.ShapeDtypeStruct((NPAD, 32), f32)] * 4,
    )(s0, s1, s2, s3, q0, q1, q2, q3, dis, br, W)


def _tc_final_call(r0, r1, r2, r3, q0, q1, q2, q3, dis, b3r, batch3, Wl, blr):
    def body(r0_r, r1_r, r2_r, r3_r, q0_r, q1_r, q2_r, q3_r, dis_r, b3_r,
             bt_r, wl_r, bl_r, out_r, sums, cnt):
        i = pl.program_id(0)

        @pl.when(i == 0)
        def _():
            sums[...] = jnp.zeros_like(sums)
            cnt[...] = jnp.zeros_like(cnt)

        t = jnp.concatenate(
            [r0_r[...] + q0_r[...], r1_r[...] + q1_r[...],
             r2_r[...] + q2_r[...], r3_r[...] + q3_r[...]], axis=1)
        h3 = t * dis_r[...] + b3_r[...]
        bt = bt_r[0]
        m = (lax.broadcasted_iota(i32, (G, BLK), 0) == bt).astype(f32)
        sums[...] += jnp.dot(m, h3, preferred_element_type=f32)
        cnt[...] += jnp.sum(m, axis=1, keepdims=True)

        @pl.when(i == NB - 1)
        def _():
            pooled = sums[...] / jnp.maximum(cnt[...], 1.0)
            out_r[...] = (jnp.dot(pooled, wl_r[...],
                                  preferred_element_type=f32) + bl_r[...])

    return pl.pallas_call(
        body,
        grid=(NB,),
        in_specs=[_rows(32)] * 8
        + [_rows(1), _full((1, H)),
           pl.BlockSpec((1, 1, BLK), lambda i: (i, 0, 0)),
           _full((H, C)), _full((1, C))],
        out_specs=pl.BlockSpec((G, C), lambda i: (0, 0)),
        out_shape=jax.ShapeDtypeStruct((G, C), f32),
        scratch_shapes=[pltpu.VMEM((G, H), f32), pltpu.VMEM((G, 1), f32)],
    )(r0, r1, r2, r3, q0, q1, q2, q3, dis, b3r, batch3, Wl, blr)


def kernel(x, edge_index, batch, W1, b1, W2, b2, W3, b3, Wl, bl):
    src = edge_index[0]
    dst = edge_index[1]
    epad = jnp.full((EPAD - E,), N, dtype=i32)
    src2d = jnp.concatenate([src, epad]).reshape(EROWS, 128)
    dst2d = jnp.concatenate([dst, epad]).reshape(EROWS, 128)

    xpad = jnp.pad(x, ((0, NPAD - N), (0, 16 - D_IN)))
    W1p = jnp.pad(W1, ((0, 16 - D_IN), (0, 0)))
    b1r = b1.reshape(1, H)
    b2r = b2.reshape(1, H)
    b3r = b3.reshape(1, H)
    blr = bl.reshape(1, C)
    batch3 = jnp.pad(batch, (0, NPAD - N),
                     constant_values=G).reshape(NB, 1, BLK)

    ones_h = jnp.ones((128,), f32)
    zer1_h = jnp.zeros((SLAB,), f32)
    zer16_h = jnp.zeros((SLAB, 16), f32)
    zer32_h = jnp.zeros((SLAB, 32), f32)

    deg0, deg1 = _deg_call(dst2d, ones_h, zer1_h)
    dis, xt = _tc_scale_call(deg0.reshape(NPAD, 1), deg1.reshape(NPAD, 1),
                             xpad)

    p0, p1 = _prop16_call(src2d, dst2d, xt, zer16_h)
    g10, g11, g12, g13 = _tc_layer1_call(p0, p1, xt, dis, W1p, b1r, W2)

    s0, s1, s2, s3 = _prop32_call(src2d, dst2d, g10, g11, g12, g13, zer32_h)
    g20, g21, g22, g23 = _tc_mid_call(s0, s1, s2, s3, g10, g11, g12, g13,
                                      dis, b2r, W3)

    r0, r1, r2, r3 = _prop32_call(src2d, dst2d, g20, g21, g22, g23, zer32_h)
    return _tc_final_call(r0, r1, r2, r3, g20, g21, g22, g23, dis, b3r,
                          batch3, Wl, blr)


# speedup vs baseline: 8.9966x; 8.9966x over previous
import functools

import jax
import jax.numpy as jnp
from jax import lax
from jax.experimental import pallas as pl
from jax.experimental.pallas import tpu as pltpu
from jax.experimental.pallas import tpu_sc as plsc

N = 50000
E = 800000
D_IN = 11
H = 128
C = 19
G = 64

NPAD = 50048
SLAB = NPAD // 16
EPAD = 819200
EROWS = EPAD // 128
BLK = 2176
NB = NPAD // BLK

_MESH = dict(core_axis_name="c", subcore_axis_name="s", num_cores=2,
             num_subcores=16)

f32 = jnp.float32
i32 = jnp.int32


def _mesh():
    return plsc.VectorSubcoreMesh(**_MESH)


def _deg_call(dst2d, ones_h, zer1_h):
    @functools.partial(
        pl.kernel,
        out_type=(jax.ShapeDtypeStruct((NPAD,), f32),
                  jax.ShapeDtypeStruct((NPAD,), f32)),
        mesh=_mesh(),
        compiler_params=pltpu.CompilerParams(use_tc_tiling_on_sc=False),
        scratch_types=[pltpu.VMEM((8, 128), i32),
                       pltpu.VMEM((128,), f32),
                       pltpu.VMEM_SHARED((NPAD,), f32)],
    )
    def k(dst_h, one_h, z_h, out0, out1, didx, ones_v, acc):
        c = lax.axis_index("c")
        s = lax.axis_index("s")
        w = c * 16 + s
        pltpu.sync_copy(one_h, ones_v)
        pltpu.sync_copy(z_h, acc.at[pl.ds(s * SLAB, SLAB)])
        plsc.subcore_barrier()

        def body(i, _):
            rb = w * 200 + i * 8
            pltpu.sync_copy(dst_h.at[pl.ds(rb, 8), :], didx)
            for j in range(8):
                pltpu.sync_copy(ones_v, acc.at[didx.at[j]], add=True)
            return 0

        lax.fori_loop(0, 25, body, 0)
        plsc.subcore_barrier()
        sl = pl.ds(s * SLAB, SLAB)

        @pl.when(c == 0)
        def _():
            pltpu.sync_copy(acc.at[sl], out0.at[sl])

        @pl.when(c == 1)
        def _():
            pltpu.sync_copy(acc.at[sl], out1.at[sl])

    return k(dst2d, ones_h, zer1_h)


def _prop16_call(src2d, dst2d, xt, zer16_h):
    @functools.partial(
        pl.kernel,
        out_type=(jax.ShapeDtypeStruct((NPAD, 16), f32),
                  jax.ShapeDtypeStruct((NPAD, 16), f32)),
        mesh=_mesh(),
        compiler_params=pltpu.CompilerParams(use_tc_tiling_on_sc=False),
        scratch_types=[pltpu.VMEM((8, 128), i32),
                       pltpu.VMEM((8, 128), i32),
                       pltpu.VMEM((128, 16), f32),
                       pltpu.VMEM_SHARED((NPAD, 16), f32)],
    )
    def k(src_h, dst_h, x_h, z_h, out0, out1, sidx, didx, rows, acc):
        c = lax.axis_index("c")
        s = lax.axis_index("s")
        w = c * 16 + s
        pltpu.sync_copy(z_h, acc.at[pl.ds(s * SLAB, SLAB), :])
        plsc.subcore_barrier()

        def body(i, _):
            rb = w * 200 + i * 8
            pltpu.sync_copy(src_h.at[pl.ds(rb, 8), :], sidx)
            pltpu.sync_copy(dst_h.at[pl.ds(rb, 8), :], didx)
            for j in range(8):
                pltpu.sync_copy(x_h.at[sidx.at[j]], rows)
                pltpu.sync_copy(rows, acc.at[didx.at[j]], add=True)
            return 0

        lax.fori_loop(0, 25, body, 0)
        plsc.subcore_barrier()
        sl = pl.ds(s * SLAB, SLAB)

        @pl.when(c == 0)
        def _():
            pltpu.sync_copy(acc.at[sl, :], out0.at[sl, :])

        @pl.when(c == 1)
        def _():
            pltpu.sync_copy(acc.at[sl, :], out1.at[sl, :])

    return k(src2d, dst2d, xt, zer16_h)


def _prop32_call(src2d, dst2d, q0, q1, q2, q3, zer32_h):
    @functools.partial(
        pl.kernel,
        out_type=tuple(jax.ShapeDtypeStruct((NPAD, 32), f32)
                       for _ in range(4)),
        mesh=_mesh(),
        compiler_params=pltpu.CompilerParams(use_tc_tiling_on_sc=False),
        scratch_types=[pltpu.VMEM((16, 128), i32),
                       pltpu.VMEM((16, 128), i32),
                       pltpu.VMEM((128, 32), f32),
                       pltpu.VMEM_SHARED((NPAD, 32), f32)],
    )
    def k(src_h, dst_h, x0, x1, x2, x3, z_h,
          o0, o1, o2, o3, sidx, didx, rows, acc):
        c = lax.axis_index("c")
        s = lax.axis_index("s")
        sl = pl.ds(s * SLAB, SLAB)

        def qpass(x_h, o_h):
            pltpu.sync_copy(z_h, acc.at[sl, :])
            plsc.subcore_barrier()

            def body(i, _):
                rb = s * 400 + i * 16
                pltpu.sync_copy(src_h.at[pl.ds(rb, 16), :], sidx)
                pltpu.sync_copy(dst_h.at[pl.ds(rb, 16), :], didx)
                for j in range(16):
                    pltpu.sync_copy(x_h.at[sidx.at[j]], rows)
                    pltpu.sync_copy(rows, acc.at[didx.at[j]], add=True)
                return 0

            lax.fori_loop(0, 25, body, 0)
            plsc.subcore_barrier()
            pltpu.sync_copy(acc.at[sl, :], o_h.at[sl, :])

        @pl.when(c == 0)
        def _():
            qpass(x0, o0)
            qpass(x1, o1)

        @pl.when(c == 1)
        def _():
            qpass(x2, o2)
            qpass(x3, o3)

    return k(src2d, dst2d, q0, q1, q2, q3, zer32_h)


def _full(shape):
    return pl.BlockSpec(shape, lambda i: (0,) * len(shape))


def _rows(w):
    return pl.BlockSpec((BLK, w), lambda i: (i, 0))


def _tc_scale_call(d0, d1, xpad):
    def body(d0_r, d1_r, x_r, dis_r, xt_r):
        dis = lax.rsqrt(d0_r[...] + d1_r[...] + 1.0)
        dis_r[...] = dis
        xt_r[...] = x_r[...] * dis

    return pl.pallas_call(
        body,
        grid=(NB,),
        in_specs=[_rows(1), _rows(1), _rows(16)],
        out_specs=[_rows(1), _rows(16)],
        out_shape=[jax.ShapeDtypeStruct((NPAD, 1), f32),
                   jax.ShapeDtypeStruct((NPAD, 16), f32)],
    )(d0, d1, xpad)


def _tc_layer1_call(p0, p1, xt, dis, W1p, b1r, W2):
    def body(p0_r, p1_r, xt_r, dis_r, w1_r, b1_r, w2_r, o0, o1, o2, o3):
        dis = dis_r[...]
        agg = (p0_r[...] + p1_r[...] + xt_r[...]) * dis
        h1 = jnp.maximum(
            jnp.dot(agg, w1_r[...], preferred_element_type=f32) + b1_r[...],
            0.0)
        g = jnp.dot(h1, w2_r[...], preferred_element_type=f32) * dis
        o0[...] = g[:, 0:32]
        o1[...] = g[:, 32:64]
        o2[...] = g[:, 64:96]
        o3[...] = g[:, 96:128]

    return pl.pallas_call(
        body,
        grid=(NB,),
        in_specs=[_rows(16), _rows(16), _rows(16), _rows(1),
                  _full((16, H)), _full((1, H)), _full((H, H))],
        out_specs=[_rows(32)] * 4,
        out_shape=[jax.ShapeDtypeStruct((NPAD, 32), f32)] * 4,
    )(p0, p1, xt, dis, W1p, b1r, W2)


def _tc_mid_call(s0, s1, s2, s3, q0, q1, q2, q3, dis, br, W):
    def body(s0_r, s1_r, s2_r, s3_r, q0_r, q1_r, q2_r, q3_r, dis_r, b_r,
             w_r, o0, o1, o2, o3):
        dis = dis_r[...]
        t = jnp.concatenate(
            [s0_r[...] + q0_r[...], s1_r[...] + q1_r[...],
             s2_r[...] + q2_r[...], s3_r[...] + q3_r[...]], axis=1)
        h = jnp.maximum(t * dis + b_r[...], 0.0)
        g = jnp.dot(h, w_r[...], preferred_element_type=f32) * dis
        o0[...] = g[:, 0:32]
        o1[...] = g[:, 32:64]
        o2[...] = g[:, 64:96]
        o3[...] = g[:, 96:128]

    return pl.pallas_call(
        body,
        grid=(NB,),
        in_specs=[_rows(32)] * 8 + [_rows(1), _full((1, H)), _full((H, H))],
        out_specs=[_rows(32)] * 4,
        out_shape=[jax.ShapeDtypeStruct((NPAD, 32), f32)] * 4,
    )(s0, s1, s2, s3, q0, q1, q2, q3, dis, br, W)


def _tc_final_call(r0, r1, r2, r3, q0, q1, q2, q3, dis, b3r, batch3, Wl, blr):
    def body(r0_r, r1_r, r2_r, r3_r, q0_r, q1_r, q2_r, q3_r, dis_r, b3_r,
             bt_r, wl_r, bl_r, out_r, sums, cnt):
        i = pl.program_id(0)

        @pl.when(i == 0)
        def _():
            sums[...] = jnp.zeros_like(sums)
            cnt[...] = jnp.zeros_like(cnt)

        t = jnp.concatenate(
            [r0_r[...] + q0_r[...], r1_r[...] + q1_r[...],
             r2_r[...] + q2_r[...], r3_r[...] + q3_r[...]], axis=1)
        h3 = t * dis_r[...] + b3_r[...]
        bt = bt_r[0]
        m = (lax.broadcasted_iota(i32, (G, BLK), 0) == bt).astype(f32)
        sums[...] += jnp.dot(m, h3, preferred_element_type=f32)
        cnt[...] += jnp.sum(m, axis=1, keepdims=True)

        @pl.when(i == NB - 1)
        def _():
            pooled = sums[...] / jnp.maximum(cnt[...], 1.0)
            out_r[...] = (jnp.dot(pooled, wl_r[...],
                                  preferred_element_type=f32) + bl_r[...])

    return pl.pallas_call(
        body,
        grid=(NB,),
        in_specs=[_rows(32)] * 8
        + [_rows(1), _full((1, H)),
           pl.BlockSpec((1, 1, BLK), lambda i: (i, 0, 0)),
           _full((H, C)), _full((1, C))],
        out_specs=pl.BlockSpec((G, C), lambda i: (0, 0)),
        out_shape=jax.ShapeDtypeStruct((G, C), f32),
        scratch_shapes=[pltpu.VMEM((G, H), f32), pltpu.VMEM((G, 1), f32)],
    )(r0, r1, r2, r3, q0, q1, q2, q3, dis, b3r, batch3, Wl, blr)


def kernel(x, edge_index, batch, W1, b1, W2, b2, W3, b3, Wl, bl):
    src = edge_index[0]
    dst = edge_index[1]
    epad = jnp.full((EPAD - E,), N, dtype=i32)
    src2d = jnp.concatenate([src, epad]).reshape(EROWS, 128)
    dst2d = jnp.concatenate([dst, epad]).reshape(EROWS, 128)

    xpad = jnp.pad(x, ((0, NPAD - N), (0, 16 - D_IN)))
    W1p = jnp.pad(W1, ((0, 16 - D_IN), (0, 0)))
    b1r = b1.reshape(1, H)
    b2r = b2.reshape(1, H)
    b3r = b3.reshape(1, H)
    blr = bl.reshape(1, C)
    batch3 = jnp.pad(batch, (0, NPAD - N),
                     constant_values=G).reshape(NB, 1, BLK)

    ones_h = jnp.ones((128,), f32)
    zer1_h = jnp.zeros((SLAB,), f32)
    zer16_h = jnp.zeros((SLAB, 16), f32)
    zer32_h = jnp.zeros((SLAB, 32), f32)

    deg0, deg1 = _deg_call(dst2d, ones_h, zer1_h)
    dis, xt = _tc_scale_call(deg0.reshape(NPAD, 1), deg1.reshape(NPAD, 1),
                             xpad)

    p0, p1 = _prop16_call(src2d, dst2d, xt, zer16_h)
    g10, g11, g12, g13 = _tc_layer1_call(p0, p1, xt, dis, W1p, b1r, W2)

    s0, s1, s2, s3 = _prop32_call(src2d, dst2d, g10, g11, g12, g13, zer32_h)
    g20, g21, g22, g23 = _tc_mid_call(s0, s1, s2, s3, g10, g11, g12, g13,
                                      dis, b2r, W3)

    r0, r1, r2, r3 = _prop32_call(src2d, dst2d, g20, g21, g22, g23, zer32_h)
    return _tc_final_call(r0, r1, r2, r3, g20, g21, g22, g23, dis, b3r,
                          batch3, Wl, blr)


# speedup vs baseline: 9.3597x; 1.0404x over previous
import functools

import jax
import jax.numpy as jnp
from jax import lax
from jax.experimental import pallas as pl
from jax.experimental.pallas import tpu as pltpu
from jax.experimental.pallas import tpu_sc as plsc

N = 50000
E = 800000
D_IN = 11
H = 128
C = 19
G = 64

NPAD = 50048
SLAB = NPAD // 16
EPAD = 819200
EROWS = EPAD // 128
BLK = 2176
NB = NPAD // BLK

_MESH = dict(core_axis_name="c", subcore_axis_name="s", num_cores=2,
             num_subcores=16)

f32 = jnp.float32
i32 = jnp.int32


def _mesh():
    return plsc.VectorSubcoreMesh(**_MESH)


def _deg_call(dst2d, ones_h, zer1_h):
    @functools.partial(
        pl.kernel,
        out_type=(jax.ShapeDtypeStruct((NPAD,), f32),
                  jax.ShapeDtypeStruct((NPAD,), f32)),
        mesh=_mesh(),
        compiler_params=pltpu.CompilerParams(use_tc_tiling_on_sc=False),
        scratch_types=[pltpu.VMEM((8, 128), i32),
                       pltpu.VMEM((128,), f32),
                       pltpu.VMEM_SHARED((NPAD,), f32)],
    )
    def k(dst_h, one_h, z_h, out0, out1, didx, ones_v, acc):
        c = lax.axis_index("c")
        s = lax.axis_index("s")
        w = c * 16 + s
        pltpu.sync_copy(one_h, ones_v)
        pltpu.sync_copy(z_h, acc.at[pl.ds(s * SLAB, SLAB)])
        plsc.subcore_barrier()

        def body(i, _):
            rb = w * 200 + i * 8
            pltpu.sync_copy(dst_h.at[pl.ds(rb, 8), :], didx)
            for j in range(8):
                pltpu.sync_copy(ones_v, acc.at[didx.at[j]], add=True)
            return 0

        lax.fori_loop(0, 25, body, 0)
        plsc.subcore_barrier()
        sl = pl.ds(s * SLAB, SLAB)

        @pl.when(c == 0)
        def _():
            pltpu.sync_copy(acc.at[sl], out0.at[sl])

        @pl.when(c == 1)
        def _():
            pltpu.sync_copy(acc.at[sl], out1.at[sl])

    return k(dst2d, ones_h, zer1_h)


def _prop16_call(src2d, dst2d, xt, zer16_h):
    @functools.partial(
        pl.kernel,
        out_type=(jax.ShapeDtypeStruct((NPAD, 16), f32),
                  jax.ShapeDtypeStruct((NPAD, 16), f32)),
        mesh=_mesh(),
        compiler_params=pltpu.CompilerParams(use_tc_tiling_on_sc=False),
        scratch_types=[pltpu.VMEM((2, 4, 128), i32),
                       pltpu.VMEM((2, 4, 128), i32),
                       pltpu.VMEM((2, 512, 16), f32),
                       pltpu.VMEM_SHARED((NPAD, 16), f32),
                       pltpu.SemaphoreType.DMA,
                       pltpu.SemaphoreType.DMA,
                       pltpu.SemaphoreType.DMA,
                       pltpu.SemaphoreType.DMA],
    )
    def k(src_h, dst_h, x_h, z_h, out0, out1, sidx, didx, rows, acc,
          sg0, sg1, ss0, ss1):
        c = lax.axis_index("c")
        s = lax.axis_index("s")
        w = c * 16 + s
        pltpu.sync_copy(z_h, acc.at[pl.ds(s * SLAB, SLAB), :])
        plsc.subcore_barrier()
        sgs = (sg0, sg1)
        sss = (ss0, ss1)

        def body(i, _):
            rb = w * 200 + i * 8
            for b in range(2):
                pltpu.sync_copy(src_h.at[pl.ds(rb + 4 * b, 4), :],
                                sidx.at[b])
                pltpu.sync_copy(dst_h.at[pl.ds(rb + 4 * b, 4), :],
                                didx.at[b])
            gd = [[pltpu.async_copy(x_h.at[sidx.at[b, j]],
                                    rows.at[b, pl.ds(j * 128, 128), :],
                                    sgs[b])
                   for j in range(4)] for b in range(2)]
            sd = []
            for b in range(2):
                for d in gd[b]:
                    d.wait()
                sd.append([pltpu.async_copy(rows.at[b, pl.ds(j * 128, 128), :],
                                            acc.at[didx.at[b, j]],
                                            sss[b], add=True)
                           for j in range(4)])
            for b in range(2):
                for d in sd[b]:
                    d.wait()
            return 0

        lax.fori_loop(0, 25, body, 0)
        plsc.subcore_barrier()
        sl = pl.ds(s * SLAB, SLAB)

        @pl.when(c == 0)
        def _():
            pltpu.sync_copy(acc.at[sl, :], out0.at[sl, :])

        @pl.when(c == 1)
        def _():
            pltpu.sync_copy(acc.at[sl, :], out1.at[sl, :])

    return k(src2d, dst2d, xt, zer16_h)


def _prop32_call(src2d, dst2d, q0, q1, q2, q3, zer32_h):
    @functools.partial(
        pl.kernel,
        out_type=tuple(jax.ShapeDtypeStruct((NPAD, 32), f32)
                       for _ in range(4)),
        mesh=_mesh(),
        compiler_params=pltpu.CompilerParams(use_tc_tiling_on_sc=False),
        scratch_types=[pltpu.VMEM((2, 2, 128), i32),
                       pltpu.VMEM((2, 2, 128), i32),
                       pltpu.VMEM((2, 256, 32), f32),
                       pltpu.VMEM_SHARED((NPAD, 32), f32),
                       pltpu.SemaphoreType.DMA,
                       pltpu.SemaphoreType.DMA,
                       pltpu.SemaphoreType.DMA,
                       pltpu.SemaphoreType.DMA],
    )
    def k(src_h, dst_h, x0, x1, x2, x3, z_h,
          o0, o1, o2, o3, sidx, didx, rows, acc, sg0, sg1, ss0, ss1):
        c = lax.axis_index("c")
        s = lax.axis_index("s")
        sl = pl.ds(s * SLAB, SLAB)
        sgs = (sg0, sg1)
        sss = (ss0, ss1)

        def qpass(x_h, o_h):
            pltpu.sync_copy(z_h, acc.at[sl, :])
            plsc.subcore_barrier()

            def body(i, _):
                rb = s * 400 + i * 4
                for b in range(2):
                    pltpu.sync_copy(src_h.at[pl.ds(rb + 2 * b, 2), :],
                                    sidx.at[b])
                    pltpu.sync_copy(dst_h.at[pl.ds(rb + 2 * b, 2), :],
                                    didx.at[b])
                gd = [[pltpu.async_copy(x_h.at[sidx.at[b, j]],
                                        rows.at[b, pl.ds(j * 128, 128), :],
                                        sgs[b])
                       for j in range(2)] for b in range(2)]
                sd = []
                for b in range(2):
                    for d in gd[b]:
                        d.wait()
                    sd.append(
                        [pltpu.async_copy(rows.at[b, pl.ds(j * 128, 128), :],
                                          acc.at[didx.at[b, j]],
                                          sss[b], add=True)
                         for j in range(2)])
                for b in range(2):
                    for d in sd[b]:
                        d.wait()
                return 0

            lax.fori_loop(0, 100, body, 0)
            plsc.subcore_barrier()
            pltpu.sync_copy(acc.at[sl, :], o_h.at[sl, :])

        @pl.when(c == 0)
        def _():
            qpass(x0, o0)
            qpass(x1, o1)

        @pl.when(c == 1)
        def _():
            qpass(x2, o2)
            qpass(x3, o3)

    return k(src2d, dst2d, q0, q1, q2, q3, zer32_h)


def _full(shape):
    return pl.BlockSpec(shape, lambda i: (0,) * len(shape))


def _rows(w):
    return pl.BlockSpec((BLK, w), lambda i: (i, 0))


def _tc_scale_call(d0, d1, xpad):
    def body(d0_r, d1_r, x_r, dis_r, xt_r):
        dis = lax.rsqrt(d0_r[...] + d1_r[...] + 1.0)
        dis_r[...] = dis
        xt_r[...] = x_r[...] * dis

    return pl.pallas_call(
        body,
        grid=(NB,),
        in_specs=[_rows(1), _rows(1), _rows(16)],
        out_specs=[_rows(1), _rows(16)],
        out_shape=[jax.ShapeDtypeStruct((NPAD, 1), f32),
                   jax.ShapeDtypeStruct((NPAD, 16), f32)],
    )(d0, d1, xpad)


def _tc_layer1_call(p0, p1, xt, dis, W1p, b1r, W2):
    def body(p0_r, p1_r, xt_r, dis_r, w1_r, b1_r, w2_r, o0, o1, o2, o3):
        dis = dis_r[...]
        agg = (p0_r[...] + p1_r[...] + xt_r[...]) * dis
        h1 = jnp.maximum(
            jnp.dot(agg, w1_r[...], preferred_element_type=f32) + b1_r[...],
            0.0)
        g = jnp.dot(h1, w2_r[...], preferred_element_type=f32) * dis
        o0[...] = g[:, 0:32]
        o1[...] = g[:, 32:64]
        o2[...] = g[:, 64:96]
        o3[...] = g[:, 96:128]

    return pl.pallas_call(
        body,
        grid=(NB,),
        in_specs=[_rows(16), _rows(16), _rows(16), _rows(1),
                  _full((16, H)), _full((1, H)), _full((H, H))],
        out_specs=[_rows(32)] * 4,
        out_shape=[jax.ShapeDtypeStruct((NPAD, 32), f32)] * 4,
    )(p0, p1, xt, dis, W1p, b1r, W2)


def _tc_mid_call(s0, s1, s2, s3, q0, q1, q2, q3, dis, br, W):
    def body(s0_r, s1_r, s2_r, s3_r, q0_r, q1_r, q2_r, q3_r, dis_r, b_r,
             w_r, o0, o1, o2, o3):
        dis = dis_r[...]
        t = jnp.concatenate(
            [s0_r[...] + q0_r[...], s1_r[...] + q1_r[...],
             s2_r[...] + q2_r[...], s3_r[...] + q3_r[...]], axis=1)
        h = jnp.maximum(t * dis + b_r[...], 0.0)
        g = jnp.dot(h, w_r[...], preferred_element_type=f32) * dis
        o0[...] = g[:, 0:32]
        o1[...] = g[:, 32:64]
        o2[...] = g[:, 64:96]
        o3[...] = g[:, 96:128]

    return pl.pallas_call(
        body,
        grid=(NB,),
        in_specs=[_rows(32)] * 8 + [_rows(1), _full((1, H)), _full((H, H))],
        out_specs=[_rows(32)] * 4,
        out_shape=[jax.ShapeDtypeStruct((NPAD, 32), f32)] * 4,
    )(s0, s1, s2, s3, q0, q1, q2, q3, dis, br, W)


def _tc_final_call(r0, r1, r2, r3, q0, q1, q2, q3, dis, b3r, batch3, Wl, blr):
    def body(r0_r, r1_r, r2_r, r3_r, q0_r, q1_r, q2_r, q3_r, dis_r, b3_r,
             bt_r, wl_r, bl_r, out_r, sums, cnt):
        i = pl.program_id(0)

        @pl.when(i == 0)
        def _():
            sums[...] = jnp.zeros_like(sums)
            cnt[...] = jnp.zeros_like(cnt)

        t = jnp.concatenate(
            [r0_r[...] + q0_r[...], r1_r[...] + q1_r[...],
             r2_r[...] + q2_r[...], r3_r[...] + q3_r[...]], axis=1)
        h3 = t * dis_r[...] + b3_r[...]
        bt = bt_r[0]
        m = (lax.broadcasted_iota(i32, (G, BLK), 0) == bt).astype(f32)
        sums[...] += jnp.dot(m, h3, preferred_element_type=f32)
        cnt[...] += jnp.sum(m, axis=1, keepdims=True)

        @pl.when(i == NB - 1)
        def _():
            pooled = sums[...] / jnp.maximum(cnt[...], 1.0)
            out_r[...] = (jnp.dot(pooled, wl_r[...],
                                  preferred_element_type=f32) + bl_r[...])

    return pl.pallas_call(
        body,
        grid=(NB,),
        in_specs=[_rows(32)] * 8
        + [_rows(1), _full((1, H)),
           pl.BlockSpec((1, 1, BLK), lambda i: (i, 0, 0)),
           _full((H, C)), _full((1, C))],
        out_specs=pl.BlockSpec((G, C), lambda i: (0, 0)),
        out_shape=jax.ShapeDtypeStruct((G, C), f32),
        scratch_shapes=[pltpu.VMEM((G, H), f32), pltpu.VMEM((G, 1), f32)],
    )(r0, r1, r2, r3, q0, q1, q2, q3, dis, b3r, batch3, Wl, blr)


def kernel(x, edge_index, batch, W1, b1, W2, b2, W3, b3, Wl, bl):
    src = edge_index[0]
    dst = edge_index[1]
    epad = jnp.full((EPAD - E,), N, dtype=i32)
    src2d = jnp.concatenate([src, epad]).reshape(EROWS, 128)
    dst2d = jnp.concatenate([dst, epad]).reshape(EROWS, 128)

    xpad = jnp.pad(x, ((0, NPAD - N), (0, 16 - D_IN)))
    W1p = jnp.pad(W1, ((0, 16 - D_IN), (0, 0)))
    b1r = b1.reshape(1, H)
    b2r = b2.reshape(1, H)
    b3r = b3.reshape(1, H)
    blr = bl.reshape(1, C)
    batch3 = jnp.pad(batch, (0, NPAD - N),
                     constant_values=G).reshape(NB, 1, BLK)

    ones_h = jnp.ones((128,), f32)
    zer1_h = jnp.zeros((SLAB,), f32)
    zer16_h = jnp.zeros((SLAB, 16), f32)
    zer32_h = jnp.zeros((SLAB, 32), f32)

    deg0, deg1 = _deg_call(dst2d, ones_h, zer1_h)
    dis, xt = _tc_scale_call(deg0.reshape(NPAD, 1), deg1.reshape(NPAD, 1),
                             xpad)

    p0, p1 = _prop16_call(src2d, dst2d, xt, zer16_h)
    g10, g11, g12, g13 = _tc_layer1_call(p0, p1, xt, dis, W1p, b1r, W2)

    s0, s1, s2, s3 = _prop32_call(src2d, dst2d, g10, g11, g12, g13, zer32_h)
    g20, g21, g22, g23 = _tc_mid_call(s0, s1, s2, s3, g10, g11, g12, g13,
                                      dis, b2r, W3)

    r0, r1, r2, r3 = _prop32_call(src2d, dst2d, g20, g21, g22, g23, zer32_h)
    return _tc_final_call(r0, r1, r2, r3, g20, g21, g22, g23, dis, b3r,
                          batch3, Wl, blr)


# speedup vs baseline: 11.5688x; 1.2360x over previous
import functools

import jax
import jax.numpy as jnp
from jax import lax
from jax.experimental import pallas as pl
from jax.experimental.pallas import tpu as pltpu
from jax.experimental.pallas import tpu_sc as plsc

N = 50000
E = 800000
D_IN = 11
H = 128
C = 19
G = 64

NPAD = 50048
SLAB = NPAD // 16
EPAD = 819200
EROWS = EPAD // 128
BLK = 2176
NB = NPAD // BLK

_MESH = dict(core_axis_name="c", subcore_axis_name="s", num_cores=2,
             num_subcores=16)

f32 = jnp.float32
i32 = jnp.int32


def _mesh():
    return plsc.VectorSubcoreMesh(**_MESH)


def _deg_call(dst2d, ones_h, zer1_h):
    @functools.partial(
        pl.kernel,
        out_type=(jax.ShapeDtypeStruct((NPAD,), f32),
                  jax.ShapeDtypeStruct((NPAD,), f32)),
        mesh=_mesh(),
        compiler_params=pltpu.CompilerParams(use_tc_tiling_on_sc=False),
        scratch_types=[pltpu.VMEM((8, 128), i32),
                       pltpu.VMEM((128,), f32),
                       pltpu.VMEM_SHARED((NPAD,), f32)],
    )
    def k(dst_h, one_h, z_h, out0, out1, didx, ones_v, acc):
        c = lax.axis_index("c")
        s = lax.axis_index("s")
        w = c * 16 + s
        pltpu.sync_copy(one_h, ones_v)
        pltpu.sync_copy(z_h, acc.at[pl.ds(s * SLAB, SLAB)])
        plsc.subcore_barrier()

        def body(i, _):
            rb = w * 200 + i * 8
            pltpu.sync_copy(dst_h.at[pl.ds(rb, 8), :], didx)
            for j in range(8):
                pltpu.sync_copy(ones_v, acc.at[didx.at[j]], add=True)
            return 0

        lax.fori_loop(0, 25, body, 0)
        plsc.subcore_barrier()
        sl = pl.ds(s * SLAB, SLAB)

        @pl.when(c == 0)
        def _():
            pltpu.sync_copy(acc.at[sl], out0.at[sl])

        @pl.when(c == 1)
        def _():
            pltpu.sync_copy(acc.at[sl], out1.at[sl])

    return k(dst2d, ones_h, zer1_h)


def _prop16_call(src2d, dst2d, xt, zer16_h):
    @functools.partial(
        pl.kernel,
        out_type=(jax.ShapeDtypeStruct((NPAD, 16), f32),
                  jax.ShapeDtypeStruct((NPAD, 16), f32)),
        mesh=_mesh(),
        compiler_params=pltpu.CompilerParams(use_tc_tiling_on_sc=False),
        scratch_types=[pltpu.VMEM((2, 4, 128), i32),
                       pltpu.VMEM((2, 4, 128), i32),
                       pltpu.VMEM((2, 512, 16), f32),
                       pltpu.VMEM_SHARED((NPAD, 16), f32),
                       pltpu.SemaphoreType.DMA,
                       pltpu.SemaphoreType.DMA,
                       pltpu.SemaphoreType.DMA,
                       pltpu.SemaphoreType.DMA],
    )
    def k(src_h, dst_h, x_h, z_h, out0, out1, sidx, didx, rows, acc,
          sg0, sg1, ss0, ss1):
        c = lax.axis_index("c")
        s = lax.axis_index("s")
        w = c * 16 + s
        pltpu.sync_copy(z_h, acc.at[pl.ds(s * SLAB, SLAB), :])
        plsc.subcore_barrier()
        sgs = (sg0, sg1)
        sss = (ss0, ss1)

        def body(i, _):
            rb = w * 200 + i * 8
            for b in range(2):
                pltpu.sync_copy(src_h.at[pl.ds(rb + 4 * b, 4), :],
                                sidx.at[b])
                pltpu.sync_copy(dst_h.at[pl.ds(rb + 4 * b, 4), :],
                                didx.at[b])
            gd = [[pltpu.async_copy(x_h.at[sidx.at[b, j]],
                                    rows.at[b, pl.ds(j * 128, 128), :],
                                    sgs[b])
                   for j in range(4)] for b in range(2)]
            sd = []
            for b in range(2):
                for d in gd[b]:
                    d.wait()
                sd.append([pltpu.async_copy(rows.at[b, pl.ds(j * 128, 128), :],
                                            acc.at[didx.at[b, j]],
                                            sss[b], add=True)
                           for j in range(4)])
            for b in range(2):
                for d in sd[b]:
                    d.wait()
            return 0

        lax.fori_loop(0, 25, body, 0)
        plsc.subcore_barrier()
        sl = pl.ds(s * SLAB, SLAB)

        @pl.when(c == 0)
        def _():
            pltpu.sync_copy(acc.at[sl, :], out0.at[sl, :])

        @pl.when(c == 1)
        def _():
            pltpu.sync_copy(acc.at[sl, :], out1.at[sl, :])

    return k(src2d, dst2d, xt, zer16_h)


def _prop32_call(src2d, dst2d, q0, q1, q2, q3, zer32_h):
    @functools.partial(
        pl.kernel,
        out_type=tuple(jax.ShapeDtypeStruct((NPAD, 32), f32)
                       for _ in range(4)),
        mesh=_mesh(),
        compiler_params=pltpu.CompilerParams(use_tc_tiling_on_sc=False),
        scratch_types=[pltpu.VMEM((2, 3, 128), i32),
                       pltpu.VMEM((2, 3, 128), i32),
                       pltpu.VMEM((2, 384, 32), f32),
                       pltpu.VMEM_SHARED((NPAD, 32), f32),
                       pltpu.SemaphoreType.DMA,
                       pltpu.SemaphoreType.DMA,
                       pltpu.SemaphoreType.DMA,
                       pltpu.SemaphoreType.DMA],
    )
    def k(src_h, dst_h, x0, x1, x2, x3, z_h,
          o0, o1, o2, o3, sidx, didx, rows, acc, sg0, sg1, ss0, ss1):
        c = lax.axis_index("c")
        s = lax.axis_index("s")
        sl = pl.ds(s * SLAB, SLAB)
        sgs = (sg0, sg1)
        sss = (ss0, ss1)

        def qpass(x_h, o_h):
            pltpu.sync_copy(z_h, acc.at[sl, :])
            plsc.subcore_barrier()

            def body(i, _):
                rb = s * 400 + i * 6
                for b in range(2):
                    @pl.when(i > 0)
                    def _(b=b):
                        for j in range(3):
                            pltpu.make_async_copy(
                                rows.at[b, pl.ds(j * 128, 128), :],
                                acc.at[didx.at[b, j]], sss[b]).wait()
                    pltpu.sync_copy(src_h.at[pl.ds(rb + 3 * b, 3), :],
                                    sidx.at[b])
                    pltpu.sync_copy(dst_h.at[pl.ds(rb + 3 * b, 3), :],
                                    didx.at[b])
                    for j in range(3):
                        pltpu.async_copy(x_h.at[sidx.at[b, j]],
                                         rows.at[b, pl.ds(j * 128, 128), :],
                                         sgs[b])
                for b in range(2):
                    for j in range(3):
                        pltpu.make_async_copy(
                            x_h.at[sidx.at[b, j]],
                            rows.at[b, pl.ds(j * 128, 128), :],
                            sgs[b]).wait()
                        pltpu.async_copy(rows.at[b, pl.ds(j * 128, 128), :],
                                         acc.at[didx.at[b, j]],
                                         sss[b], add=True)
                return 0

            lax.fori_loop(0, 66, body, 0)

            def tail(b, j):
                rb = s * 400 + 396 + 2 * b + j
                pltpu.sync_copy(src_h.at[pl.ds(rb, 1), :],
                                sidx.at[b, pl.ds(j, 1), :])
                pltpu.sync_copy(dst_h.at[pl.ds(rb, 1), :],
                                didx.at[b, pl.ds(j, 1), :])
                pltpu.sync_copy(x_h.at[sidx.at[b, j]],
                                rows.at[b, pl.ds(j * 128, 128), :])
                pltpu.sync_copy(rows.at[b, pl.ds(j * 128, 128), :],
                                acc.at[didx.at[b, j]], add=True)

            for b in range(2):
                for j in range(3):
                    pltpu.make_async_copy(
                        rows.at[b, pl.ds(j * 128, 128), :],
                        acc.at[didx.at[b, j]], sss[b]).wait()
            for b in range(2):
                for j in range(2):
                    tail(b, j)
            plsc.subcore_barrier()
            pltpu.sync_copy(acc.at[sl, :], o_h.at[sl, :])

        @pl.when(c == 0)
        def _():
            qpass(x0, o0)
            qpass(x1, o1)

        @pl.when(c == 1)
        def _():
            qpass(x2, o2)
            qpass(x3, o3)

    return k(src2d, dst2d, q0, q1, q2, q3, zer32_h)


def _full(shape):
    return pl.BlockSpec(shape, lambda i: (0,) * len(shape))


def _rows(w):
    return pl.BlockSpec((BLK, w), lambda i: (i, 0))


def _tc_scale_call(d0, d1, xpad):
    def body(d0_r, d1_r, x_r, dis_r, xt_r):
        dis = lax.rsqrt(d0_r[...] + d1_r[...] + 1.0)
        dis_r[...] = dis
        xt_r[...] = x_r[...] * dis

    return pl.pallas_call(
        body,
        grid=(NB,),
        in_specs=[_rows(1), _rows(1), _rows(16)],
        out_specs=[_rows(1), _rows(16)],
        out_shape=[jax.ShapeDtypeStruct((NPAD, 1), f32),
                   jax.ShapeDtypeStruct((NPAD, 16), f32)],
    )(d0, d1, xpad)


def _tc_layer1_call(p0, p1, xt, dis, W1p, b1r, W2):
    def body(p0_r, p1_r, xt_r, dis_r, w1_r, b1_r, w2_r, o0, o1, o2, o3):
        dis = dis_r[...]
        agg = (p0_r[...] + p1_r[...] + xt_r[...]) * dis
        h1 = jnp.maximum(
            jnp.dot(agg, w1_r[...], preferred_element_type=f32) + b1_r[...],
            0.0)
        g = jnp.dot(h1, w2_r[...], preferred_element_type=f32) * dis
        o0[...] = g[:, 0:32]
        o1[...] = g[:, 32:64]
        o2[...] = g[:, 64:96]
        o3[...] = g[:, 96:128]

    return pl.pallas_call(
        body,
        grid=(NB,),
        in_specs=[_rows(16), _rows(16), _rows(16), _rows(1),
                  _full((16, H)), _full((1, H)), _full((H, H))],
        out_specs=[_rows(32)] * 4,
        out_shape=[jax.ShapeDtypeStruct((NPAD, 32), f32)] * 4,
    )(p0, p1, xt, dis, W1p, b1r, W2)


def _tc_mid_call(s0, s1, s2, s3, q0, q1, q2, q3, dis, br, W):
    def body(s0_r, s1_r, s2_r, s3_r, q0_r, q1_r, q2_r, q3_r, dis_r, b_r,
             w_r, o0, o1, o2, o3):
        dis = dis_r[...]
        t = jnp.concatenate(
            [s0_r[...] + q0_r[...], s1_r[...] + q1_r[...],
             s2_r[...] + q2_r[...], s3_r[...] + q3_r[...]], axis=1)
        h = jnp.maximum(t * dis + b_r[...], 0.0)
        g = jnp.dot(h, w_r[...], preferred_element_type=f32) * dis
        o0[...] = g[:, 0:32]
        o1[...] = g[:, 32:64]
        o2[...] = g[:, 64:96]
        o3[...] = g[:, 96:128]

    return pl.pallas_call(
        body,
        grid=(NB,),
        in_specs=[_rows(32)] * 8 + [_rows(1), _full((1, H)), _full((H, H))],
        out_specs=[_rows(32)] * 4,
        out_shape=[jax.ShapeDtypeStruct((NPAD, 32), f32)] * 4,
    )(s0, s1, s2, s3, q0, q1, q2, q3, dis, br, W)


def _tc_final_call(r0, r1, r2, r3, q0, q1, q2, q3, dis, b3r, batch3, Wl, blr):
    def body(r0_r, r1_r, r2_r, r3_r, q0_r, q1_r, q2_r, q3_r, dis_r, b3_r,
             bt_r, wl_r, bl_r, out_r, sums, cnt):
        i = pl.program_id(0)

        @pl.when(i == 0)
        def _():
            sums[...] = jnp.zeros_like(sums)
            cnt[...] = jnp.zeros_like(cnt)

        t = jnp.concatenate(
            [r0_r[...] + q0_r[...], r1_r[...] + q1_r[...],
             r2_r[...] + q2_r[...], r3_r[...] + q3_r[...]], axis=1)
        h3 = t * dis_r[...] + b3_r[...]
        bt = bt_r[0]
        m = (lax.broadcasted_iota(i32, (G, BLK), 0) == bt).astype(f32)
        sums[...] += jnp.dot(m, h3, preferred_element_type=f32)
        cnt[...] += jnp.sum(m, axis=1, keepdims=True)

        @pl.when(i == NB - 1)
        def _():
            pooled = sums[...] / jnp.maximum(cnt[...], 1.0)
            out_r[...] = (jnp.dot(pooled, wl_r[...],
                                  preferred_element_type=f32) + bl_r[...])

    return pl.pallas_call(
        body,
        grid=(NB,),
        in_specs=[_rows(32)] * 8
        + [_rows(1), _full((1, H)),
           pl.BlockSpec((1, 1, BLK), lambda i: (i, 0, 0)),
           _full((H, C)), _full((1, C))],
        out_specs=pl.BlockSpec((G, C), lambda i: (0, 0)),
        out_shape=jax.ShapeDtypeStruct((G, C), f32),
        scratch_shapes=[pltpu.VMEM((G, H), f32), pltpu.VMEM((G, 1), f32)],
    )(r0, r1, r2, r3, q0, q1, q2, q3, dis, b3r, batch3, Wl, blr)


def kernel(x, edge_index, batch, W1, b1, W2, b2, W3, b3, Wl, bl):
    src = edge_index[0]
    dst = edge_index[1]
    epad = jnp.full((EPAD - E,), N, dtype=i32)
    src2d = jnp.concatenate([src, epad]).reshape(EROWS, 128)
    dst2d = jnp.concatenate([dst, epad]).reshape(EROWS, 128)

    xpad = jnp.pad(x, ((0, NPAD - N), (0, 16 - D_IN)))
    W1p = jnp.pad(W1, ((0, 16 - D_IN), (0, 0)))
    b1r = b1.reshape(1, H)
    b2r = b2.reshape(1, H)
    b3r = b3.reshape(1, H)
    blr = bl.reshape(1, C)
    batch3 = jnp.pad(batch, (0, NPAD - N),
                     constant_values=G).reshape(NB, 1, BLK)

    ones_h = jnp.ones((128,), f32)
    zer1_h = jnp.zeros((SLAB,), f32)
    zer16_h = jnp.zeros((SLAB, 16), f32)
    zer32_h = jnp.zeros((SLAB, 32), f32)

    deg0, deg1 = _deg_call(dst2d, ones_h, zer1_h)
    dis, xt = _tc_scale_call(deg0.reshape(NPAD, 1), deg1.reshape(NPAD, 1),
                             xpad)

    p0, p1 = _prop16_call(src2d, dst2d, xt, zer16_h)
    g10, g11, g12, g13 = _tc_layer1_call(p0, p1, xt, dis, W1p, b1r, W2)

    s0, s1, s2, s3 = _prop32_call(src2d, dst2d, g10, g11, g12, g13, zer32_h)
    g20, g21, g22, g23 = _tc_mid_call(s0, s1, s2, s3, g10, g11, g12, g13,
                                      dis, b2r, W3)

    r0, r1, r2, r3 = _prop32_call(src2d, dst2d, g20, g21, g22, g23, zer32_h)
    return _tc_final_call(r0, r1, r2, r3, g20, g21, g22, g23, dis, b3r,
                          batch3, Wl, blr)


# speedup vs baseline: 12.3555x; 1.0680x over previous
import functools

import jax
import jax.numpy as jnp
from jax import lax
from jax.experimental import pallas as pl
from jax.experimental.pallas import tpu as pltpu
from jax.experimental.pallas import tpu_sc as plsc

N = 50000
E = 800000
D_IN = 11
H = 128
C = 19
G = 64

NPAD = 50048
SLAB = NPAD // 16
EPAD = 819200
EROWS = EPAD // 128
BLK = 2176
NB = NPAD // BLK

_MESH = dict(core_axis_name="c", subcore_axis_name="s", num_cores=2,
             num_subcores=16)

f32 = jnp.float32
i32 = jnp.int32


def _mesh():
    return plsc.VectorSubcoreMesh(**_MESH)


def _deg_call(dst2d, ones_h, zer1_h):
    @functools.partial(
        pl.kernel,
        out_type=(jax.ShapeDtypeStruct((NPAD,), f32),
                  jax.ShapeDtypeStruct((NPAD,), f32)),
        mesh=_mesh(),
        compiler_params=pltpu.CompilerParams(use_tc_tiling_on_sc=False),
        scratch_types=[pltpu.VMEM((8, 128), i32),
                       pltpu.VMEM((128,), f32),
                       pltpu.VMEM_SHARED((NPAD,), f32)],
    )
    def k(dst_h, one_h, z_h, out0, out1, didx, ones_v, acc):
        c = lax.axis_index("c")
        s = lax.axis_index("s")
        w = c * 16 + s
        pltpu.sync_copy(one_h, ones_v)
        pltpu.sync_copy(z_h, acc.at[pl.ds(s * SLAB, SLAB)])
        plsc.subcore_barrier()

        def body(i, _):
            rb = w * 200 + i * 8
            pltpu.sync_copy(dst_h.at[pl.ds(rb, 8), :], didx)
            for j in range(8):
                pltpu.sync_copy(ones_v, acc.at[didx.at[j]], add=True)
            return 0

        lax.fori_loop(0, 25, body, 0)
        plsc.subcore_barrier()
        sl = pl.ds(s * SLAB, SLAB)

        @pl.when(c == 0)
        def _():
            pltpu.sync_copy(acc.at[sl], out0.at[sl])

        @pl.when(c == 1)
        def _():
            pltpu.sync_copy(acc.at[sl], out1.at[sl])

    return k(dst2d, ones_h, zer1_h)


def _prop16_call(src2d, dst2d, xt, zer16_h):
    @functools.partial(
        pl.kernel,
        out_type=(jax.ShapeDtypeStruct((NPAD, 16), f32),
                  jax.ShapeDtypeStruct((NPAD, 16), f32)),
        mesh=_mesh(),
        compiler_params=pltpu.CompilerParams(use_tc_tiling_on_sc=False),
        scratch_types=[pltpu.VMEM((2, 4, 128), i32),
                       pltpu.VMEM((2, 4, 128), i32),
                       pltpu.VMEM((2, 512, 16), f32),
                       pltpu.VMEM_SHARED((NPAD, 16), f32),
                       pltpu.SemaphoreType.DMA,
                       pltpu.SemaphoreType.DMA,
                       pltpu.SemaphoreType.DMA,
                       pltpu.SemaphoreType.DMA,
                       pltpu.SemaphoreType.DMA,
                       pltpu.SemaphoreType.DMA],
    )
    def k(src_h, dst_h, x_h, z_h, out0, out1, sidx, didx, rows, acc,
          sg0, sg1, ss0, ss1, si0, si1):
        c = lax.axis_index("c")
        s = lax.axis_index("s")
        w = c * 16 + s
        pltpu.sync_copy(z_h, acc.at[pl.ds(s * SLAB, SLAB), :])
        plsc.subcore_barrier()
        sgs = (sg0, sg1)
        sss = (ss0, ss1)
        sis = (si0, si1)

        def body(i, _):
            rb = w * 200 + i * 8
            for b in range(2):
                @pl.when(i > 0)
                def _(b=b):
                    for j in range(4):
                        pltpu.make_async_copy(
                            rows.at[b, pl.ds(j * 128, 128), :],
                            acc.at[didx.at[b, j]], sss[b]).wait()
                pltpu.async_copy(src_h.at[pl.ds(rb + 4 * b, 4), :],
                                 sidx.at[b], sis[b])
                pltpu.async_copy(dst_h.at[pl.ds(rb + 4 * b, 4), :],
                                 didx.at[b], sis[b])
            for b in range(2):
                pltpu.make_async_copy(src_h.at[pl.ds(rb + 4 * b, 4), :],
                                      sidx.at[b], sis[b]).wait()
                pltpu.make_async_copy(dst_h.at[pl.ds(rb + 4 * b, 4), :],
                                      didx.at[b], sis[b]).wait()
                for j in range(4):
                    pltpu.async_copy(x_h.at[sidx.at[b, j]],
                                     rows.at[b, pl.ds(j * 128, 128), :],
                                     sgs[b])
            for b in range(2):
                for j in range(4):
                    pltpu.make_async_copy(
                        x_h.at[sidx.at[b, j]],
                        rows.at[b, pl.ds(j * 128, 128), :], sgs[b]).wait()
                    pltpu.async_copy(rows.at[b, pl.ds(j * 128, 128), :],
                                     acc.at[didx.at[b, j]],
                                     sss[b], add=True)
            return 0

        lax.fori_loop(0, 25, body, 0)
        for b in range(2):
            for j in range(4):
                pltpu.make_async_copy(
                    rows.at[b, pl.ds(j * 128, 128), :],
                    acc.at[didx.at[b, j]], sss[b]).wait()
        plsc.subcore_barrier()
        sl = pl.ds(s * SLAB, SLAB)

        @pl.when(c == 0)
        def _():
            pltpu.sync_copy(acc.at[sl, :], out0.at[sl, :])

        @pl.when(c == 1)
        def _():
            pltpu.sync_copy(acc.at[sl, :], out1.at[sl, :])

    return k(src2d, dst2d, xt, zer16_h)


def _prop32_call(src2d, dst2d, q0, q1, q2, q3, zer32_h):
    @functools.partial(
        pl.kernel,
        out_type=tuple(jax.ShapeDtypeStruct((NPAD, 32), f32)
                       for _ in range(4)),
        mesh=_mesh(),
        compiler_params=pltpu.CompilerParams(use_tc_tiling_on_sc=False),
        scratch_types=[pltpu.VMEM((2, 3, 128), i32),
                       pltpu.VMEM((2, 3, 128), i32),
                       pltpu.VMEM((2, 384, 32), f32),
                       pltpu.VMEM_SHARED((NPAD, 32), f32),
                       pltpu.SemaphoreType.DMA,
                       pltpu.SemaphoreType.DMA,
                       pltpu.SemaphoreType.DMA,
                       pltpu.SemaphoreType.DMA,
                       pltpu.SemaphoreType.DMA,
                       pltpu.SemaphoreType.DMA],
    )
    def k(src_h, dst_h, x0, x1, x2, x3, z_h,
          o0, o1, o2, o3, sidx, didx, rows, acc, sg0, sg1, ss0, ss1,
          si0, si1):
        c = lax.axis_index("c")
        s = lax.axis_index("s")
        sl = pl.ds(s * SLAB, SLAB)
        sgs = (sg0, sg1)
        sss = (ss0, ss1)
        sis = (si0, si1)

        def qpass(x_h, o_h):
            pltpu.sync_copy(z_h, acc.at[sl, :])
            plsc.subcore_barrier()

            def body(i, _):
                rb = s * 400 + i * 6
                for b in range(2):
                    @pl.when(i > 0)
                    def _(b=b):
                        for j in range(3):
                            pltpu.make_async_copy(
                                rows.at[b, pl.ds(j * 128, 128), :],
                                acc.at[didx.at[b, j]], sss[b]).wait()
                    pltpu.async_copy(src_h.at[pl.ds(rb + 3 * b, 3), :],
                                     sidx.at[b], sis[b])
                    pltpu.async_copy(dst_h.at[pl.ds(rb + 3 * b, 3), :],
                                     didx.at[b], sis[b])
                for b in range(2):
                    pltpu.make_async_copy(src_h.at[pl.ds(rb + 3 * b, 3), :],
                                          sidx.at[b], sis[b]).wait()
                    pltpu.make_async_copy(dst_h.at[pl.ds(rb + 3 * b, 3), :],
                                          didx.at[b], sis[b]).wait()
                    for j in range(3):
                        pltpu.async_copy(x_h.at[sidx.at[b, j]],
                                         rows.at[b, pl.ds(j * 128, 128), :],
                                         sgs[b])
                for b in range(2):
                    for j in range(3):
                        pltpu.make_async_copy(
                            x_h.at[sidx.at[b, j]],
                            rows.at[b, pl.ds(j * 128, 128), :],
                            sgs[b]).wait()
                        pltpu.async_copy(rows.at[b, pl.ds(j * 128, 128), :],
                                         acc.at[didx.at[b, j]],
                                         sss[b], add=True)
                return 0

            lax.fori_loop(0, 66, body, 0)

            def tail(b, j):
                rb = s * 400 + 396 + 2 * b + j
                pltpu.sync_copy(src_h.at[pl.ds(rb, 1), :],
                                sidx.at[b, pl.ds(j, 1), :])
                pltpu.sync_copy(dst_h.at[pl.ds(rb, 1), :],
                                didx.at[b, pl.ds(j, 1), :])
                pltpu.sync_copy(x_h.at[sidx.at[b, j]],
                                rows.at[b, pl.ds(j * 128, 128), :])
                pltpu.sync_copy(rows.at[b, pl.ds(j * 128, 128), :],
                                acc.at[didx.at[b, j]], add=True)

            for b in range(2):
                for j in range(3):
                    pltpu.make_async_copy(
                        rows.at[b, pl.ds(j * 128, 128), :],
                        acc.at[didx.at[b, j]], sss[b]).wait()
            for b in range(2):
                for j in range(2):
                    tail(b, j)
            plsc.subcore_barrier()
            pltpu.sync_copy(acc.at[sl, :], o_h.at[sl, :])

        @pl.when(c == 0)
        def _():
            qpass(x0, o0)
            qpass(x1, o1)

        @pl.when(c == 1)
        def _():
            qpass(x2, o2)
            qpass(x3, o3)

    return k(src2d, dst2d, q0, q1, q2, q3, zer32_h)


def _full(shape):
    return pl.BlockSpec(shape, lambda i: (0,) * len(shape))


def _rows(w):
    return pl.BlockSpec((BLK, w), lambda i: (i, 0))


def _tc_scale_call(d0, d1, xpad):
    def body(d0_r, d1_r, x_r, dis_r, xt_r):
        dis = lax.rsqrt(d0_r[...] + d1_r[...] + 1.0)
        dis_r[...] = dis
        xt_r[...] = x_r[...] * dis

    return pl.pallas_call(
        body,
        grid=(NB,),
        in_specs=[_rows(1), _rows(1), _rows(16)],
        out_specs=[_rows(1), _rows(16)],
        out_shape=[jax.ShapeDtypeStruct((NPAD, 1), f32),
                   jax.ShapeDtypeStruct((NPAD, 16), f32)],
    )(d0, d1, xpad)


def _tc_layer1_call(p0, p1, xt, dis, W1p, b1r, W2):
    def body(p0_r, p1_r, xt_r, dis_r, w1_r, b1_r, w2_r, o0, o1, o2, o3):
        dis = dis_r[...]
        agg = (p0_r[...] + p1_r[...] + xt_r[...]) * dis
        h1 = jnp.maximum(
            jnp.dot(agg, w1_r[...], preferred_element_type=f32) + b1_r[...],
            0.0)
        g = jnp.dot(h1, w2_r[...], preferred_element_type=f32) * dis
        o0[...] = g[:, 0:32]
        o1[...] = g[:, 32:64]
        o2[...] = g[:, 64:96]
        o3[...] = g[:, 96:128]

    return pl.pallas_call(
        body,
        grid=(NB,),
        in_specs=[_rows(16), _rows(16), _rows(16), _rows(1),
                  _full((16, H)), _full((1, H)), _full((H, H))],
        out_specs=[_rows(32)] * 4,
        out_shape=[jax.ShapeDtypeStruct((NPAD, 32), f32)] * 4,
    )(p0, p1, xt, dis, W1p, b1r, W2)


def _tc_mid_call(s0, s1, s2, s3, q0, q1, q2, q3, dis, br, W):
    def body(s0_r, s1_r, s2_r, s3_r, q0_r, q1_r, q2_r, q3_r, dis_r, b_r,
             w_r, o0, o1, o2, o3):
        dis = dis_r[...]
        t = jnp.concatenate(
            [s0_r[...] + q0_r[...], s1_r[...] + q1_r[...],
             s2_r[...] + q2_r[...], s3_r[...] + q3_r[...]], axis=1)
        h = jnp.maximum(t * dis + b_r[...], 0.0)
        g = jnp.dot(h, w_r[...], preferred_element_type=f32) * dis
        o0[...] = g[:, 0:32]
        o1[...] = g[:, 32:64]
        o2[...] = g[:, 64:96]
        o3[...] = g[:, 96:128]

    return pl.pallas_call(
        body,
        grid=(NB,),
        in_specs=[_rows(32)] * 8 + [_rows(1), _full((1, H)), _full((H, H))],
        out_specs=[_rows(32)] * 4,
        out_shape=[jax.ShapeDtypeStruct((NPAD, 32), f32)] * 4,
    )(s0, s1, s2, s3, q0, q1, q2, q3, dis, br, W)


def _tc_final_call(r0, r1, r2, r3, q0, q1, q2, q3, dis, b3r, batch3, Wl, blr):
    def body(r0_r, r1_r, r2_r, r3_r, q0_r, q1_r, q2_r, q3_r, dis_r, b3_r,
             bt_r, wl_r, bl_r, out_r, sums, cnt):
        i = pl.program_id(0)

        @pl.when(i == 0)
        def _():
            sums[...] = jnp.zeros_like(sums)
            cnt[...] = jnp.zeros_like(cnt)

        t = jnp.concatenate(
            [r0_r[...] + q0_r[...], r1_r[...] + q1_r[...],
             r2_r[...] + q2_r[...], r3_r[...] + q3_r[...]], axis=1)
        h3 = t * dis_r[...] + b3_r[...]
        bt = bt_r[0]
        m = (lax.broadcasted_iota(i32, (G, BLK), 0) == bt).astype(f32)
        sums[...] += jnp.dot(m, h3, preferred_element_type=f32)
        cnt[...] += jnp.sum(m, axis=1, keepdims=True)

        @pl.when(i == NB - 1)
        def _():
            pooled = sums[...] / jnp.maximum(cnt[...], 1.0)
            out_r[...] = (jnp.dot(pooled, wl_r[...],
                                  preferred_element_type=f32) + bl_r[...])

    return pl.pallas_call(
        body,
        grid=(NB,),
        in_specs=[_rows(32)] * 8
        + [_rows(1), _full((1, H)),
           pl.BlockSpec((1, 1, BLK), lambda i: (i, 0, 0)),
           _full((H, C)), _full((1, C))],
        out_specs=pl.BlockSpec((G, C), lambda i: (0, 0)),
        out_shape=jax.ShapeDtypeStruct((G, C), f32),
        scratch_shapes=[pltpu.VMEM((G, H), f32), pltpu.VMEM((G, 1), f32)],
    )(r0, r1, r2, r3, q0, q1, q2, q3, dis, b3r, batch3, Wl, blr)


def kernel(x, edge_index, batch, W1, b1, W2, b2, W3, b3, Wl, bl):
    src = edge_index[0]
    dst = edge_index[1]
    epad = jnp.full((EPAD - E,), N, dtype=i32)
    src2d = jnp.concatenate([src, epad]).reshape(EROWS, 128)
    dst2d = jnp.concatenate([dst, epad]).reshape(EROWS, 128)

    xpad = jnp.pad(x, ((0, NPAD - N), (0, 16 - D_IN)))
    W1p = jnp.pad(W1, ((0, 16 - D_IN), (0, 0)))
    b1r = b1.reshape(1, H)
    b2r = b2.reshape(1, H)
    b3r = b3.reshape(1, H)
    blr = bl.reshape(1, C)
    batch3 = jnp.pad(batch, (0, NPAD - N),
                     constant_values=G).reshape(NB, 1, BLK)

    ones_h = jnp.ones((128,), f32)
    zer1_h = jnp.zeros((SLAB,), f32)
    zer16_h = jnp.zeros((SLAB, 16), f32)
    zer32_h = jnp.zeros((SLAB, 32), f32)

    deg0, deg1 = _deg_call(dst2d, ones_h, zer1_h)
    dis, xt = _tc_scale_call(deg0.reshape(NPAD, 1), deg1.reshape(NPAD, 1),
                             xpad)

    p0, p1 = _prop16_call(src2d, dst2d, xt, zer16_h)
    g10, g11, g12, g13 = _tc_layer1_call(p0, p1, xt, dis, W1p, b1r, W2)

    s0, s1, s2, s3 = _prop32_call(src2d, dst2d, g10, g11, g12, g13, zer32_h)
    g20, g21, g22, g23 = _tc_mid_call(s0, s1, s2, s3, g10, g11, g12, g13,
                                      dis, b2r, W3)

    r0, r1, r2, r3 = _prop32_call(src2d, dst2d, g20, g21, g22, g23, zer32_h)
    return _tc_final_call(r0, r1, r2, r3, g20, g21, g22, g23, dis, b3r,
                          batch3, Wl, blr)


# speedup vs baseline: 12.3781x; 1.0018x over previous
import functools

import jax
import jax.numpy as jnp
from jax import lax
from jax.experimental import pallas as pl
from jax.experimental.pallas import tpu as pltpu
from jax.experimental.pallas import tpu_sc as plsc

N = 50000
E = 800000
D_IN = 11
H = 128
C = 19
G = 64

NPAD = 50048
SLAB = NPAD // 16
EPAD = 819200
EROWS = EPAD // 128
BLK = 3128
NB = NPAD // BLK

_MESH = dict(core_axis_name="c", subcore_axis_name="s", num_cores=2,
             num_subcores=16)

f32 = jnp.float32
i32 = jnp.int32


def _mesh():
    return plsc.VectorSubcoreMesh(**_MESH)


def _deg_call(dst2d, ones_h, zer1_h):
    @functools.partial(
        pl.kernel,
        out_type=(jax.ShapeDtypeStruct((NPAD,), f32),
                  jax.ShapeDtypeStruct((NPAD,), f32)),
        mesh=_mesh(),
        compiler_params=pltpu.CompilerParams(use_tc_tiling_on_sc=False),
        scratch_types=[pltpu.VMEM((8, 128), i32),
                       pltpu.VMEM((128,), f32),
                       pltpu.VMEM_SHARED((NPAD,), f32)],
    )
    def k(dst_h, one_h, z_h, out0, out1, didx, ones_v, acc):
        c = lax.axis_index("c")
        s = lax.axis_index("s")
        w = c * 16 + s
        pltpu.sync_copy(one_h, ones_v)
        pltpu.sync_copy(z_h, acc.at[pl.ds(s * SLAB, SLAB)])
        plsc.subcore_barrier()

        def body(i, _):
            rb = w * 200 + i * 8
            pltpu.sync_copy(dst_h.at[pl.ds(rb, 8), :], didx)
            for j in range(8):
                pltpu.sync_copy(ones_v, acc.at[didx.at[j]], add=True)
            return 0

        lax.fori_loop(0, 25, body, 0)
        plsc.subcore_barrier()
        sl = pl.ds(s * SLAB, SLAB)

        @pl.when(c == 0)
        def _():
            pltpu.sync_copy(acc.at[sl], out0.at[sl])

        @pl.when(c == 1)
        def _():
            pltpu.sync_copy(acc.at[sl], out1.at[sl])

    return k(dst2d, ones_h, zer1_h)


def _prop16_call(src2d, dst2d, xt, zer16_h):
    @functools.partial(
        pl.kernel,
        out_type=(jax.ShapeDtypeStruct((NPAD, 16), f32),
                  jax.ShapeDtypeStruct((NPAD, 16), f32)),
        mesh=_mesh(),
        compiler_params=pltpu.CompilerParams(use_tc_tiling_on_sc=False),
        scratch_types=[pltpu.VMEM((2, 4, 128), i32),
                       pltpu.VMEM((2, 4, 128), i32),
                       pltpu.VMEM((2, 512, 16), f32),
                       pltpu.VMEM_SHARED((NPAD, 16), f32),
                       pltpu.SemaphoreType.DMA,
                       pltpu.SemaphoreType.DMA,
                       pltpu.SemaphoreType.DMA,
                       pltpu.SemaphoreType.DMA,
                       pltpu.SemaphoreType.DMA,
                       pltpu.SemaphoreType.DMA],
    )
    def k(src_h, dst_h, x_h, z_h, out0, out1, sidx, didx, rows, acc,
          sg0, sg1, ss0, ss1, si0, si1):
        c = lax.axis_index("c")
        s = lax.axis_index("s")
        w = c * 16 + s
        pltpu.sync_copy(z_h, acc.at[pl.ds(s * SLAB, SLAB), :])
        plsc.subcore_barrier()
        sgs = (sg0, sg1)
        sss = (ss0, ss1)
        sis = (si0, si1)

        def body(i, _):
            rb = w * 200 + i * 8
            for b in range(2):
                @pl.when(i > 0)
                def _(b=b):
                    for j in range(4):
                        pltpu.make_async_copy(
                            rows.at[b, pl.ds(j * 128, 128), :],
                            acc.at[didx.at[b, j]], sss[b]).wait()
                pltpu.async_copy(src_h.at[pl.ds(rb + 4 * b, 4), :],
                                 sidx.at[b], sis[b])
                pltpu.async_copy(dst_h.at[pl.ds(rb + 4 * b, 4), :],
                                 didx.at[b], sis[b])
            for b in range(2):
                pltpu.make_async_copy(src_h.at[pl.ds(rb + 4 * b, 4), :],
                                      sidx.at[b], sis[b]).wait()
                pltpu.make_async_copy(dst_h.at[pl.ds(rb + 4 * b, 4), :],
                                      didx.at[b], sis[b]).wait()
                for j in range(4):
                    pltpu.async_copy(x_h.at[sidx.at[b, j]],
                                     rows.at[b, pl.ds(j * 128, 128), :],
                                     sgs[b])
            for b in range(2):
                for j in range(4):
                    pltpu.make_async_copy(
                        x_h.at[sidx.at[b, j]],
                        rows.at[b, pl.ds(j * 128, 128), :], sgs[b]).wait()
                    pltpu.async_copy(rows.at[b, pl.ds(j * 128, 128), :],
                                     acc.at[didx.at[b, j]],
                                     sss[b], add=True)
            return 0

        lax.fori_loop(0, 25, body, 0)
        for b in range(2):
            for j in range(4):
                pltpu.make_async_copy(
                    rows.at[b, pl.ds(j * 128, 128), :],
                    acc.at[didx.at[b, j]], sss[b]).wait()
        plsc.subcore_barrier()
        sl = pl.ds(s * SLAB, SLAB)

        @pl.when(c == 0)
        def _():
            pltpu.sync_copy(acc.at[sl, :], out0.at[sl, :])

        @pl.when(c == 1)
        def _():
            pltpu.sync_copy(acc.at[sl, :], out1.at[sl, :])

    return k(src2d, dst2d, xt, zer16_h)


def _prop32_call(src2d, dst2d, q0, q1, q2, q3, zer32_h):
    @functools.partial(
        pl.kernel,
        out_type=tuple(jax.ShapeDtypeStruct((NPAD, 32), f32)
                       for _ in range(4)),
        mesh=_mesh(),
        compiler_params=pltpu.CompilerParams(use_tc_tiling_on_sc=False),
        scratch_types=[pltpu.VMEM((2, 3, 128), i32),
                       pltpu.VMEM((2, 3, 128), i32),
                       pltpu.VMEM((2, 384, 32), f32),
                       pltpu.VMEM_SHARED((NPAD, 32), f32),
                       pltpu.SemaphoreType.DMA,
                       pltpu.SemaphoreType.DMA,
                       pltpu.SemaphoreType.DMA,
                       pltpu.SemaphoreType.DMA,
                       pltpu.SemaphoreType.DMA,
                       pltpu.SemaphoreType.DMA],
    )
    def k(src_h, dst_h, x0, x1, x2, x3, z_h,
          o0, o1, o2, o3, sidx, didx, rows, acc, sg0, sg1, ss0, ss1,
          si0, si1):
        c = lax.axis_index("c")
        s = lax.axis_index("s")
        sl = pl.ds(s * SLAB, SLAB)
        sgs = (sg0, sg1)
        sss = (ss0, ss1)
        sis = (si0, si1)

        def qpass(x_h, o_h):
            pltpu.sync_copy(z_h, acc.at[sl, :])
            plsc.subcore_barrier()

            def body(i, _):
                rb = s * 400 + i * 6
                for b in range(2):
                    @pl.when(i > 0)
                    def _(b=b):
                        for j in range(3):
                            pltpu.make_async_copy(
                                rows.at[b, pl.ds(j * 128, 128), :],
                                acc.at[didx.at[b, j]], sss[b]).wait()
                    pltpu.async_copy(src_h.at[pl.ds(rb + 3 * b, 3), :],
                                     sidx.at[b], sis[b])
                    pltpu.async_copy(dst_h.at[pl.ds(rb + 3 * b, 3), :],
                                     didx.at[b], sis[b])
                for b in range(2):
                    pltpu.make_async_copy(src_h.at[pl.ds(rb + 3 * b, 3), :],
                                          sidx.at[b], sis[b]).wait()
                    pltpu.make_async_copy(dst_h.at[pl.ds(rb + 3 * b, 3), :],
                                          didx.at[b], sis[b]).wait()
                    for j in range(3):
                        pltpu.async_copy(x_h.at[sidx.at[b, j]],
                                         rows.at[b, pl.ds(j * 128, 128), :],
                                         sgs[b])
                for b in range(2):
                    for j in range(3):
                        pltpu.make_async_copy(
                            x_h.at[sidx.at[b, j]],
                            rows.at[b, pl.ds(j * 128, 128), :],
                            sgs[b]).wait()
                        pltpu.async_copy(rows.at[b, pl.ds(j * 128, 128), :],
                                         acc.at[didx.at[b, j]],
                                         sss[b], add=True)
                return 0

            lax.fori_loop(0, 66, body, 0)

            def tail(b, j):
                rb = s * 400 + 396 + 2 * b + j
                pltpu.sync_copy(src_h.at[pl.ds(rb, 1), :],
                                sidx.at[b, pl.ds(j, 1), :])
                pltpu.sync_copy(dst_h.at[pl.ds(rb, 1), :],
                                didx.at[b, pl.ds(j, 1), :])
                pltpu.sync_copy(x_h.at[sidx.at[b, j]],
                                rows.at[b, pl.ds(j * 128, 128), :])
                pltpu.sync_copy(rows.at[b, pl.ds(j * 128, 128), :],
                                acc.at[didx.at[b, j]], add=True)

            for b in range(2):
                for j in range(3):
                    pltpu.make_async_copy(
                        rows.at[b, pl.ds(j * 128, 128), :],
                        acc.at[didx.at[b, j]], sss[b]).wait()
            for b in range(2):
                for j in range(2):
                    tail(b, j)
            plsc.subcore_barrier()
            pltpu.sync_copy(acc.at[sl, :], o_h.at[sl, :])

        @pl.when(c == 0)
        def _():
            qpass(x0, o0)
            qpass(x1, o1)

        @pl.when(c == 1)
        def _():
            qpass(x2, o2)
            qpass(x3, o3)

    return k(src2d, dst2d, q0, q1, q2, q3, zer32_h)


def _full(shape):
    return pl.BlockSpec(shape, lambda i: (0,) * len(shape))


def _rows(w):
    return pl.BlockSpec((BLK, w), lambda i: (i, 0))


def _tc_scale_call(d0, d1, xpad):
    def body(d0_r, d1_r, x_r, dis_r, xt_r):
        dis = lax.rsqrt(d0_r[...] + d1_r[...] + 1.0)
        dis_r[...] = dis
        xt_r[...] = x_r[...] * dis

    return pl.pallas_call(
        body,
        grid=(NB,),
        in_specs=[_rows(1), _rows(1), _rows(16)],
        out_specs=[_rows(1), _rows(16)],
        out_shape=[jax.ShapeDtypeStruct((NPAD, 1), f32),
                   jax.ShapeDtypeStruct((NPAD, 16), f32)],
    )(d0, d1, xpad)


def _tc_layer1_call(p0, p1, xt, dis, W1p, b1r, W2):
    def body(p0_r, p1_r, xt_r, dis_r, w1_r, b1_r, w2_r, o0, o1, o2, o3):
        dis = dis_r[...]
        agg = (p0_r[...] + p1_r[...] + xt_r[...]) * dis
        h1 = jnp.maximum(
            jnp.dot(agg, w1_r[...], preferred_element_type=f32) + b1_r[...],
            0.0)
        g = jnp.dot(h1, w2_r[...], preferred_element_type=f32) * dis
        o0[...] = g[:, 0:32]
        o1[...] = g[:, 32:64]
        o2[...] = g[:, 64:96]
        o3[...] = g[:, 96:128]

    return pl.pallas_call(
        body,
        grid=(NB,),
        in_specs=[_rows(16), _rows(16), _rows(16), _rows(1),
                  _full((16, H)), _full((1, H)), _full((H, H))],
        out_specs=[_rows(32)] * 4,
        out_shape=[jax.ShapeDtypeStruct((NPAD, 32), f32)] * 4,
    )(p0, p1, xt, dis, W1p, b1r, W2)


def _tc_mid_call(s0, s1, s2, s3, q0, q1, q2, q3, dis, br, W):
    def body(s0_r, s1_r, s2_r, s3_r, q0_r, q1_r, q2_r, q3_r, dis_r, b_r,
             w_r, o0, o1, o2, o3):
        dis = dis_r[...]
        t = jnp.concatenate(
            [s0_r[...] + q0_r[...], s1_r[...] + q1_r[...],
             s2_r[...] + q2_r[...], s3_r[...] + q3_r[...]], axis=1)
        h = jnp.maximum(t * dis + b_r[...], 0.0)
        g = jnp.dot(h, w_r[...], preferred_element_type=f32) * dis
        o0[...] = g[:, 0:32]
        o1[...] = g[:, 32:64]
        o2[...] = g[:, 64:96]
        o3[...] = g[:, 96:128]

    return pl.pallas_call(
        body,
        grid=(NB,),
        in_specs=[_rows(32)] * 8 + [_rows(1), _full((1, H)), _full((H, H))],
        out_specs=[_rows(32)] * 4,
        out_shape=[jax.ShapeDtypeStruct((NPAD, 32), f32)] * 4,
    )(s0, s1, s2, s3, q0, q1, q2, q3, dis, br, W)


def _tc_final_call(r0, r1, r2, r3, q0, q1, q2, q3, dis, b3r, batch3, Wl, blr):
    def body(r0_r, r1_r, r2_r, r3_r, q0_r, q1_r, q2_r, q3_r, dis_r, b3_r,
             bt_r, wl_r, bl_r, out_r, sums, cnt):
        i = pl.program_id(0)

        @pl.when(i == 0)
        def _():
            sums[...] = jnp.zeros_like(sums)
            cnt[...] = jnp.zeros_like(cnt)

        t = jnp.concatenate(
            [r0_r[...] + q0_r[...], r1_r[...] + q1_r[...],
             r2_r[...] + q2_r[...], r3_r[...] + q3_r[...]], axis=1)
        h3 = t * dis_r[...] + b3_r[...]
        bt = bt_r[0]
        m = (lax.broadcasted_iota(i32, (G, BLK), 0) == bt).astype(f32)
        sums[...] += jnp.dot(m, h3, preferred_element_type=f32)
        cnt[...] += jnp.sum(m, axis=1, keepdims=True)

        @pl.when(i == NB - 1)
        def _():
            pooled = sums[...] / jnp.maximum(cnt[...], 1.0)
            out_r[...] = (jnp.dot(pooled, wl_r[...],
                                  preferred_element_type=f32) + bl_r[...])

    return pl.pallas_call(
        body,
        grid=(NB,),
        in_specs=[_rows(32)] * 8
        + [_rows(1), _full((1, H)),
           pl.BlockSpec((1, 1, BLK), lambda i: (i, 0, 0)),
           _full((H, C)), _full((1, C))],
        out_specs=pl.BlockSpec((G, C), lambda i: (0, 0)),
        out_shape=jax.ShapeDtypeStruct((G, C), f32),
        scratch_shapes=[pltpu.VMEM((G, H), f32), pltpu.VMEM((G, 1), f32)],
    )(r0, r1, r2, r3, q0, q1, q2, q3, dis, b3r, batch3, Wl, blr)


def kernel(x, edge_index, batch, W1, b1, W2, b2, W3, b3, Wl, bl):
    src = edge_index[0]
    dst = edge_index[1]
    epad = jnp.full((EPAD - E,), N, dtype=i32)
    src2d = jnp.concatenate([src, epad]).reshape(EROWS, 128)
    dst2d = jnp.concatenate([dst, epad]).reshape(EROWS, 128)

    xpad = jnp.pad(x, ((0, NPAD - N), (0, 16 - D_IN)))
    W1p = jnp.pad(W1, ((0, 16 - D_IN), (0, 0)))
    b1r = b1.reshape(1, H)
    b2r = b2.reshape(1, H)
    b3r = b3.reshape(1, H)
    blr = bl.reshape(1, C)
    batch3 = jnp.pad(batch, (0, NPAD - N),
                     constant_values=G).reshape(NB, 1, BLK)

    ones_h = jnp.ones((128,), f32)
    zer1_h = jnp.zeros((SLAB,), f32)
    zer16_h = jnp.zeros((SLAB, 16), f32)
    zer32_h = jnp.zeros((SLAB, 32), f32)

    deg0, deg1 = _deg_call(dst2d, ones_h, zer1_h)
    dis, xt = _tc_scale_call(deg0.reshape(NPAD, 1), deg1.reshape(NPAD, 1),
                             xpad)

    p0, p1 = _prop16_call(src2d, dst2d, xt, zer16_h)
    g10, g11, g12, g13 = _tc_layer1_call(p0, p1, xt, dis, W1p, b1r, W2)

    s0, s1, s2, s3 = _prop32_call(src2d, dst2d, g10, g11, g12, g13, zer32_h)
    g20, g21, g22, g23 = _tc_mid_call(s0, s1, s2, s3, g10, g11, g12, g13,
                                      dis, b2r, W3)

    r0, r1, r2, r3 = _prop32_call(src2d, dst2d, g20, g21, g22, g23, zer32_h)
    return _tc_final_call(r0, r1, r2, r3, g20, g21, g22, g23, dis, b3r,
                          batch3, Wl, blr)
